# baseline (speedup 1.0000x reference)
import functools

import jax
import jax.numpy as jnp
from jax import lax
from jax.experimental import pallas as pl
from jax.experimental.pallas import tpu as pltpu
from jax.experimental.pallas import tpu_sc as plsc

NC = 2
NS = 16
LANES = 16
BLK = 128
NBUF = 2
RB = 2048

_mesh = plsc.VectorSubcoreMesh(core_axis_name="c", subcore_axis_name="s")


def _pad_edges(src, dst, n, npad):
    e = src.shape[0]
    grp = 2 * NBUF * BLK
    ept = -(-e // (NS * grp)) * grp
    pad = NS * ept - e
    k = jnp.arange(pad, dtype=jnp.int32)
    src_p = jnp.concatenate([src, k % 64]).reshape(NS, ept // BLK, BLK)
    dst_p = jnp.concatenate([dst, n + k % (npad - n)])
    src2 = jnp.stack([src_p, src_p + n])
    return src2, dst_p.reshape(NS, ept // BLK, BLK), ept // BLK


def _deg_kernel(dst2, rpt, npad):
    rows_per_tile = npad // NS

    @functools.partial(
        pl.kernel,
        out_type=jax.ShapeDtypeStruct((NC, npad), jnp.float32),
        mesh=_mesh,
        scratch_types=[
            pltpu.VMEM((rpt, BLK), jnp.int32),
            pltpu.VMEM((rows_per_tile,), jnp.float32),
            pltpu.VMEM((BLK,), jnp.float32),
            pltpu.VMEM_SHARED((npad,), jnp.float32),
        ],
    )
    def k(dst_hbm, out_hbm, idx_v, zb_v, ones_v, acc_sh):
        cid = lax.axis_index("c")
        sid = lax.axis_index("s")

        @pl.loop(0, rows_per_tile, step=LANES)
        def _(i):
            zb_v[pl.ds(i, LANES)] = jnp.zeros((LANES,), jnp.float32)

        @pl.loop(0, BLK, step=LANES)
        def _(i):
            ones_v[pl.ds(i, LANES)] = jnp.ones((LANES,), jnp.float32)

        pltpu.sync_copy(zb_v, acc_sh.at[pl.ds(sid * rows_per_tile, rows_per_tile)])
        plsc.subcore_barrier()

        pltpu.sync_copy(dst_hbm.at[sid], idx_v)

        @pl.loop(0, rpt)
        def _(j):
            @pl.when(lax.rem(j, 2) == cid)
            def _():
                pltpu.sync_copy(ones_v, acc_sh.at[idx_v.at[j]], add=True)

        plsc.subcore_barrier()
        pltpu.sync_copy(
            acc_sh.at[pl.ds(sid * rows_per_tile, rows_per_tile)],
            out_hbm.at[cid, pl.ds(sid * rows_per_tile, rows_per_tile)],
        )

    return k(dst2)


def _edge_kernel(g2n, src2, dst2, rpt, npad):
    dh = g2n.shape[1]
    rows_per_tile = npad // NS
    zchunks = rows_per_tile // BLK

    @functools.partial(
        pl.kernel,
        out_type=jax.ShapeDtypeStruct((NC, npad, dh), jnp.float32),
        mesh=_mesh,
        compiler_params=pltpu.CompilerParams(use_tc_tiling_on_sc=False),
        scratch_types=[
            pltpu.VMEM((rpt, BLK), jnp.int32),
            pltpu.VMEM((rpt, BLK), jnp.int32),
            pltpu.VMEM((2 * NBUF, BLK, dh), jnp.float32),
            pltpu.VMEM_SHARED((npad, dh), jnp.float32),
            pltpu.SemaphoreType.DMA,
            pltpu.SemaphoreType.DMA,
        ],
    )
    def k(g_hbm, src_hbm, dst_hbm, out_hbm, src_v, dst_v, rows_v, acc_sh,
          gsem, ssem):
        cid = lax.axis_index("c")
        sid = lax.axis_index("s")

        zb_v = rows_v.at[0]

        @pl.loop(0, BLK)
        def _(r):
            @pl.loop(0, dh, step=LANES)
            def _(c2):
                zb_v[r, pl.ds(c2, LANES)] = jnp.zeros((LANES,), jnp.float32)

        @pl.loop(0, zchunks)
        def _(q):
            pltpu.sync_copy(zb_v, acc_sh.at[pl.ds(sid * rows_per_tile + q * BLK, BLK)])

        plsc.subcore_barrier()

        pltpu.sync_copy(src_hbm.at[cid, sid], src_v)
        pltpu.sync_copy(dst_hbm.at[sid], dst_v)

        def fire_g(j0, s):
            for b in range(NBUF):
                pltpu.async_copy(g_hbm.at[src_v.at[j0 + b]],
                                 rows_v.at[s * NBUF + b], gsem)

        def wait_g(j0, s):
            for b in range(NBUF):
                pltpu.make_async_copy(g_hbm.at[src_v.at[j0 + b]],
                                      rows_v.at[s * NBUF + b], gsem).wait()

        def fire_s(j0, s):
            for b in range(NBUF):
                pltpu.async_copy(rows_v.at[s * NBUF + b],
                                 acc_sh.at[dst_v.at[j0 + b]], ssem, add=True)

        def wait_s(j0, s):
            for b in range(NBUF):
                pltpu.make_async_copy(rows_v.at[s * NBUF + b],
                                      acc_sh.at[dst_v.at[j0 + b]], ssem).wait()

        ngrp = rpt // NBUF
        fire_g(0, 0)

        @pl.loop(0, ngrp, step=2)
        def _(t):
            j0 = t * NBUF
            wait_g(j0, 0)
            fire_g(j0 + NBUF, 1)
            fire_s(j0, 0)
            wait_s(j0, 0)

            @pl.when(t + 2 < ngrp)
            def _():
                fire_g(j0 + 2 * NBUF, 0)

            wait_g(j0 + NBUF, 1)
            fire_s(j0 + NBUF, 1)
            wait_s(j0 + NBUF, 1)

        plsc.subcore_barrier()

        @pl.loop(0, zchunks)
        def _(q):
            r0 = sid * rows_per_tile + q * BLK
            pltpu.sync_copy(acc_sh.at[pl.ds(r0, BLK)], out_hbm.at[cid, pl.ds(r0, BLK)])

    return k(g2n, src2, dst2)


def _matmul_tc(x, w):
    n, d = x.shape

    def body(x_ref, w_ref, o_ref):
        o_ref[...] = jnp.dot(x_ref[...], w_ref[...],
                             preferred_element_type=jnp.float32,
                             precision=lax.Precision.HIGHEST)

    return pl.pallas_call(
        body,
        grid=(pl.cdiv(n, RB),),
        in_specs=[pl.BlockSpec((RB, d), lambda i: (i, 0)),
                  pl.BlockSpec((d, w.shape[1]), lambda i: (0, 0))],
        out_specs=pl.BlockSpec((RB, w.shape[1]), lambda i: (i, 0)),
        out_shape=jax.ShapeDtypeStruct((n, w.shape[1]), jnp.float32),
    )(x, w)


def _scale_tc(degp, h):
    n, d = h.shape
    dh = d // 2

    def body(deg_ref, h_ref, g_ref, dinv_ref):
        deg = deg_ref[0, :] + deg_ref[1, :] + 1.0
        dinv = lax.rsqrt(deg)
        g = h_ref[...] * dinv[:, None]
        g_ref[0] = g[:, :dh]
        g_ref[1] = g[:, dh:]
        dinv_ref[...] = dinv[:, None]

    return pl.pallas_call(
        body,
        grid=(pl.cdiv(n, RB),),
        in_specs=[pl.BlockSpec((NC, RB), lambda i: (0, i)),
                  pl.BlockSpec((RB, d), lambda i: (i, 0))],
        out_specs=[pl.BlockSpec((NC, RB, dh), lambda i: (0, i, 0)),
                   pl.BlockSpec((RB, 1), lambda i: (i, 0))],
        out_shape=[jax.ShapeDtypeStruct((NC, n, dh), jnp.float32),
                   jax.ShapeDtypeStruct((n, 1), jnp.float32)],
    )(degp, h)


def _mid_tc(s, g, dinv, b, w):
    _, n, dh = g.shape
    d = 2 * dh

    def body(s_ref, g_ref, dinv_ref, b_ref, w_ref, o_ref):
        t = jnp.concatenate([s_ref[0] + g_ref[0], s_ref[1] + g_ref[1]], axis=1)
        t = t * dinv_ref[...] + b_ref[...]
        h = jnp.maximum(t, 0.0)
        y = jnp.dot(h, w_ref[...], preferred_element_type=jnp.float32,
                    precision=lax.Precision.HIGHEST) * dinv_ref[...]
        o_ref[0] = y[:, :dh]
        o_ref[1] = y[:, dh:]

    return pl.pallas_call(
        body,
        grid=(pl.cdiv(n, RB),),
        in_specs=[pl.BlockSpec((NC, RB, dh), lambda i: (0, i, 0)),
                  pl.BlockSpec((NC, RB, dh), lambda i: (0, i, 0)),
                  pl.BlockSpec((RB, 1), lambda i: (i, 0)),
                  pl.BlockSpec((1, d), lambda i: (0, 0)),
                  pl.BlockSpec((d, d), lambda i: (0, 0))],
        out_specs=pl.BlockSpec((NC, RB, dh), lambda i: (0, i, 0)),
        out_shape=jax.ShapeDtypeStruct((NC, n, dh), jnp.float32),
    )(s, g, dinv, b.reshape(1, d), w)


def _final_tc(s, g, dinv, b):
    _, n, dh = g.shape
    d = 2 * dh

    def body(s_ref, g_ref, dinv_ref, b_ref, o_ref):
        t = jnp.concatenate([s_ref[0] + g_ref[0], s_ref[1] + g_ref[1]], axis=1)
        o_ref[...] = t * dinv_ref[...] + b_ref[...]

    return pl.pallas_call(
        body,
        grid=(pl.cdiv(n, RB),),
        in_specs=[pl.BlockSpec((NC, RB, dh), lambda i: (0, i, 0)),
                  pl.BlockSpec((NC, RB, dh), lambda i: (0, i, 0)),
                  pl.BlockSpec((RB, 1), lambda i: (i, 0)),
                  pl.BlockSpec((1, d), lambda i: (0, 0))],
        out_specs=pl.BlockSpec((RB, d), lambda i: (i, 0)),
        out_shape=jax.ShapeDtypeStruct((n, d), jnp.float32),
    )(s, g, dinv, b.reshape(1, d))


def kernel(x, edge_index, W1, b1, W2, b2):
    n, d = x.shape
    npad = -(-(n + 1) // (NS * BLK)) * (NS * BLK)

    src2, dst2, rpt = _pad_edges(edge_index[0], edge_index[1], n, npad)

    degp = _deg_kernel(dst2, rpt, npad)
    h1 = _matmul_tc(x, W1)
    g1, dinv = _scale_tc(degp, h1)

    s1 = _edge_kernel(g1.reshape(NC * n, d // 2), src2, dst2, rpt, npad)
    g2 = _mid_tc(s1, g1, dinv, b1, W2)

    s2 = _edge_kernel(g2.reshape(NC * n, d // 2), src2, dst2, rpt, npad)
    return _final_tc(s2, g2, dinv, b2)


# speedup vs baseline: 1.3184x; 1.3184x over previous
import functools

import jax
import jax.numpy as jnp
from jax import lax
from jax.experimental import pallas as pl
from jax.experimental.pallas import tpu as pltpu
from jax.experimental.pallas import tpu_sc as plsc

NC = 2
NS = 16
LANES = 16
BLK = 128
KBUF = 6
LAG = 3
RB = 2048

_mesh = plsc.VectorSubcoreMesh(core_axis_name="c", subcore_axis_name="s")


def _pad_edges(src, dst, n, npad):
    e = src.shape[0]
    ept = -(-e // (NS * BLK)) * BLK
    pad = NS * ept - e
    k = jnp.arange(pad, dtype=jnp.int32)
    src_p = jnp.concatenate([src, k % 64]).reshape(NS, ept // BLK, BLK)
    dst_p = jnp.concatenate([dst, n + k % (npad - n)])
    src2 = jnp.stack([2 * src_p, 2 * src_p + 1])
    return src2, dst_p.reshape(NS, ept // BLK, BLK), ept // BLK


def _deg_kernel(dst2, rpt, npad):
    rows_per_tile = npad // NS

    @functools.partial(
        pl.kernel,
        out_type=jax.ShapeDtypeStruct((NC, npad), jnp.float32),
        mesh=_mesh,
        scratch_types=[
            pltpu.VMEM((rpt, BLK), jnp.int32),
            pltpu.VMEM((rows_per_tile,), jnp.float32),
            pltpu.VMEM((BLK,), jnp.float32),
            pltpu.VMEM_SHARED((npad,), jnp.float32),
        ],
    )
    def k(dst_hbm, out_hbm, idx_v, zb_v, ones_v, acc_sh):
        cid = lax.axis_index("c")
        sid = lax.axis_index("s")

        @pl.loop(0, rows_per_tile, step=LANES)
        def _(i):
            zb_v[pl.ds(i, LANES)] = jnp.zeros((LANES,), jnp.float32)

        @pl.loop(0, BLK, step=LANES)
        def _(i):
            ones_v[pl.ds(i, LANES)] = jnp.ones((LANES,), jnp.float32)

        pltpu.sync_copy(zb_v, acc_sh.at[pl.ds(sid * rows_per_tile, rows_per_tile)])
        plsc.subcore_barrier()

        pltpu.sync_copy(dst_hbm.at[sid], idx_v)

        @pl.loop(0, rpt)
        def _(j):
            @pl.when(lax.rem(j, 2) == cid)
            def _():
                pltpu.sync_copy(ones_v, acc_sh.at[idx_v.at[j]], add=True)

        plsc.subcore_barrier()
        pltpu.sync_copy(
            acc_sh.at[pl.ds(sid * rows_per_tile, rows_per_tile)],
            out_hbm.at[cid, pl.ds(sid * rows_per_tile, rows_per_tile)],
        )

    return k(dst2)


def _edge_kernel(g2n, src2, dst2, rpt, npad):
    dh = g2n.shape[1]
    rows_per_tile = npad // NS
    zchunks = rows_per_tile // BLK

    @functools.partial(
        pl.kernel,
        out_type=jax.ShapeDtypeStruct((NC, npad, 2 * dh), jnp.float32),
        mesh=_mesh,
        compiler_params=pltpu.CompilerParams(use_tc_tiling_on_sc=False),
        scratch_types=[
            pltpu.VMEM((rpt, BLK), jnp.int32),
            pltpu.VMEM((rpt, BLK), jnp.int32),
            pltpu.VMEM((KBUF, BLK, dh), jnp.float32),
            pltpu.VMEM_SHARED((npad, dh), jnp.float32),
            pltpu.SemaphoreType.DMA,
            pltpu.SemaphoreType.DMA,
            pltpu.SemaphoreType.DMA,
        ],
    )
    def k(g_hbm, src_hbm, dst_hbm, out_hbm, src_v, dst_v, rows_v, acc_sh,
          gsem, ssem, hsem):
        cid = lax.axis_index("c")
        sid = lax.axis_index("s")

        pltpu.async_copy(src_hbm.at[cid, sid], src_v, hsem)
        pltpu.async_copy(dst_hbm.at[sid], dst_v, hsem)

        zb_v = rows_v.at[0]

        @pl.loop(0, BLK)
        def _(r):
            @pl.loop(0, dh, step=LANES)
            def _(c2):
                zb_v[r, pl.ds(c2, LANES)] = jnp.zeros((LANES,), jnp.float32)

        @pl.loop(0, zchunks)
        def _(q):
            pltpu.async_copy(zb_v, acc_sh.at[pl.ds(sid * rows_per_tile + q * BLK, BLK)],
                             ssem)

        @pl.loop(0, zchunks)
        def _(q):
            pltpu.make_async_copy(zb_v,
                                  acc_sh.at[pl.ds(sid * rows_per_tile + q * BLK, BLK)],
                                  ssem).wait()
        pltpu.make_async_copy(src_hbm.at[cid, sid], src_v, hsem).wait()
        pltpu.make_async_copy(dst_hbm.at[sid], dst_v, hsem).wait()

        plsc.subcore_barrier()

        def fire_g(j):
            pltpu.async_copy(g_hbm.at[src_v.at[j]],
                             rows_v.at[lax.rem(j, KBUF)], gsem)

        def wait_g(j):
            pltpu.make_async_copy(g_hbm.at[src_v.at[j]],
                                  rows_v.at[lax.rem(j, KBUF)], gsem).wait()

        def fire_s(j):
            pltpu.async_copy(rows_v.at[lax.rem(j, KBUF)],
                             acc_sh.at[dst_v.at[j]], ssem, add=True)

        def wait_s(j):
            pltpu.make_async_copy(rows_v.at[lax.rem(j, KBUF)],
                                  acc_sh.at[dst_v.at[j]], ssem).wait()

        @pl.loop(0, KBUF)
        def _(p):
            fire_g(p)

        @pl.loop(0, rpt)
        def _(j):
            wait_g(j)
            fire_s(j)

            @pl.when(j >= LAG)
            def _():
                wait_s(j - LAG)

                @pl.when(j + KBUF - LAG < rpt)
                def _():
                    fire_g(j + KBUF - LAG)

        @pl.loop(rpt - LAG, rpt)
        def _(j):
            wait_s(j)

        plsc.subcore_barrier()

        @pl.loop(0, zchunks)
        def _(q):
            r0 = sid * rows_per_tile + q * BLK
            pltpu.async_copy(acc_sh.at[pl.ds(r0, BLK)],
                             out_hbm.at[cid, pl.ds(r0, BLK), pl.ds(0, dh)], hsem)

        @pl.loop(0, zchunks)
        def _(q):
            r0 = sid * rows_per_tile + q * BLK
            pltpu.make_async_copy(acc_sh.at[pl.ds(r0, BLK)],
                                  out_hbm.at[cid, pl.ds(r0, BLK), pl.ds(0, dh)],
                                  hsem).wait()

    return k(g2n, src2, dst2)


def _layer1_tc(degp, x, w):
    n, d = x.shape
    dh = d // 2

    def body(deg_ref, x_ref, w_ref, g_ref, dinv_ref):
        deg = deg_ref[0, :] + deg_ref[1, :] + 1.0
        dinv = lax.rsqrt(deg)
        h = jnp.dot(x_ref[...], w_ref[...],
                    preferred_element_type=jnp.float32,
                    precision=lax.Precision.HIGHEST)
        g_ref[...] = h * dinv[:, None]
        dinv_ref[...] = dinv[:, None]

    return pl.pallas_call(
        body,
        grid=(pl.cdiv(n, RB),),
        in_specs=[pl.BlockSpec((NC, RB), lambda i: (0, i)),
                  pl.BlockSpec((RB, d), lambda i: (i, 0)),
                  pl.BlockSpec((d, d), lambda i: (0, 0))],
        out_specs=[pl.BlockSpec((RB, d), lambda i: (i, 0)),
                   pl.BlockSpec((RB, 1), lambda i: (i, 0))],
        out_shape=[jax.ShapeDtypeStruct((n, d), jnp.float32),
                   jax.ShapeDtypeStruct((n, 1), jnp.float32)],
    )(degp, x, w)


def _mid_tc(s, g, dinv, b, w):
    n, d = g.shape
    dh = d // 2

    def body(s_ref, g_ref, dinv_ref, b_ref, w_ref, o_ref):
        sfull = jnp.concatenate([s_ref[0][:, :dh], s_ref[1][:, :dh]], axis=1)
        t = (sfull + g_ref[...]) * dinv_ref[...] + b_ref[...]
        h = jnp.maximum(t, 0.0)
        o_ref[...] = jnp.dot(h, w_ref[...], preferred_element_type=jnp.float32,
                             precision=lax.Precision.HIGHEST) * dinv_ref[...]

    return pl.pallas_call(
        body,
        grid=(pl.cdiv(n, RB),),
        in_specs=[pl.BlockSpec((NC, RB, d), lambda i: (0, i, 0)),
                  pl.BlockSpec((RB, d), lambda i: (i, 0)),
                  pl.BlockSpec((RB, 1), lambda i: (i, 0)),
                  pl.BlockSpec((1, d), lambda i: (0, 0)),
                  pl.BlockSpec((d, d), lambda i: (0, 0))],
        out_specs=pl.BlockSpec((RB, d), lambda i: (i, 0)),
        out_shape=jax.ShapeDtypeStruct((n, d), jnp.float32),
    )(s, g, dinv, b.reshape(1, d), w)


def _final_tc(s, g, dinv, b):
    n, d = g.shape
    dh = d // 2

    def body(s_ref, g_ref, dinv_ref, b_ref, o_ref):
        sfull = jnp.concatenate([s_ref[0][:, :dh], s_ref[1][:, :dh]], axis=1)
        o_ref[...] = (sfull + g_ref[...]) * dinv_ref[...] + b_ref[...]

    return pl.pallas_call(
        body,
        grid=(pl.cdiv(n, RB),),
        in_specs=[pl.BlockSpec((NC, RB, d), lambda i: (0, i, 0)),
                  pl.BlockSpec((RB, d), lambda i: (i, 0)),
                  pl.BlockSpec((RB, 1), lambda i: (i, 0)),
                  pl.BlockSpec((1, d), lambda i: (0, 0))],
        out_specs=pl.BlockSpec((RB, d), lambda i: (i, 0)),
        out_shape=jax.ShapeDtypeStruct((n, d), jnp.float32),
    )(s, g, dinv, b.reshape(1, d))


def kernel(x, edge_index, W1, b1, W2, b2):
    n, d = x.shape
    npad = -(-(n + 1) // (NS * BLK)) * (NS * BLK)

    src2, dst2, rpt = _pad_edges(edge_index[0], edge_index[1], n, npad)

    degp = _deg_kernel(dst2, rpt, npad)
    g1, dinv = _layer1_tc(degp, x, W1)

    dh = d // 2
    s1 = _edge_kernel(g1.reshape(2 * n, dh), src2, dst2, rpt, npad)
    g2 = _mid_tc(s1, g1, dinv, b1, W2)

    s2 = _edge_kernel(g2.reshape(2 * n, dh), src2, dst2, rpt, npad)
    return _final_tc(s2, g2, dinv, b2)
